# baseline (speedup 1.0000x reference)
import functools

import jax
import jax.numpy as jnp
from jax import lax
from jax.experimental import pallas as pl
from jax.experimental.pallas import tpu as pltpu
from jax.experimental.pallas import tpu_sc as plsc

_N = 10000
_E = 320000
_NC = 2
_NS = 16
_CH = 80


def _make_agg(D):
    rows_per_tile = 624
    tail_rows = _N - _NS * rows_per_tile
    tail_r0 = _NS * rows_per_tile
    n_phase = 5
    n_chunk = 25
    mesh = plsc.VectorSubcoreMesh(core_axis_name="c", subcore_axis_name="s")

    @functools.partial(
        pl.kernel,
        out_type=jax.ShapeDtypeStruct((_NC, _N, D), jnp.float32),
        mesh=mesh,
        scratch_types=[
            pltpu.VMEM((2, n_chunk, _CH), jnp.int32),
            pltpu.VMEM((2, n_chunk, _CH), jnp.int32),
            pltpu.VMEM((_CH, D), jnp.float32),
            pltpu.VMEM((_CH, D), jnp.float32),
            pltpu.VMEM_SHARED((_N, D), jnp.float32),
            pltpu.SemaphoreType.DMA,
            pltpu.SemaphoreType.DMA,
            pltpu.SemaphoreType.DMA,
            pltpu.SemaphoreType.DMA,
            pltpu.SemaphoreType.DMA,
        ],
    )
    def agg(x_hbm, src_hbm, dst_hbm, out_hbm, srcs, dsts, buf0, buf1, acc,
            gs0, gs1, ss0, ss1, ssem):
        c = lax.axis_index("c")
        s = lax.axis_index("s")
        w = c * _NS + s
        r0 = pl.multiple_of(s * rows_per_tile, 8)
        pltpu.sync_copy(src_hbm.at[w, 0], srcs.at[0])
        pltpu.sync_copy(dst_hbm.at[w, 0], dsts.at[0])
        pltpu.sync_copy(x_hbm.at[pl.ds(r0, rows_per_tile)],
                        acc.at[pl.ds(r0, rows_per_tile)])

        @pl.when(s == 0)
        def _seed_tail():
            pltpu.sync_copy(x_hbm.at[pl.ds(tail_r0, tail_rows)],
                            acc.at[pl.ds(tail_r0, tail_rows)])

        plsc.subcore_barrier()

        def gather(pb, j, buf, sem):
            return pltpu.async_copy(x_hbm.at[srcs.at[pb, j]], buf, sem)

        def gwait(pb, j, buf, sem):
            pltpu.make_async_copy(x_hbm.at[srcs.at[pb, j]], buf, sem).wait()

        def scat(pb, j, buf, sem):
            return pltpu.async_copy(buf, acc.at[dsts.at[pb, j]], sem,
                                    add=True)

        def swait(pb, j, buf, sem):
            pltpu.make_async_copy(buf, acc.at[dsts.at[pb, j]], sem).wait()

        for p in range(n_phase):
            pb = p % 2
            if p + 1 < n_phase:
                pltpu.async_copy(src_hbm.at[w, p + 1], srcs.at[1 - pb], ssem)
                pltpu.async_copy(dst_hbm.at[w, p + 1], dsts.at[1 - pb], ssem)

            gather(pb, 0, buf0, gs0)
            gather(pb, 1, buf1, gs1)

            def body(i, carry, pb=pb):
                j = 2 * i
                gwait(pb, j, buf0, gs0)
                scat(pb, j, buf0, ss0)
                gwait(pb, j + 1, buf1, gs1)
                scat(pb, j + 1, buf1, ss1)
                swait(pb, j, buf0, ss0)
                gather(pb, j + 2, buf0, gs0)
                swait(pb, j + 1, buf1, ss1)
                gather(pb, j + 3, buf1, gs1)
                return carry

            lax.fori_loop(0, (n_chunk - 3) // 2, body, 0)
            gwait(pb, n_chunk - 3, buf0, gs0)
            scat(pb, n_chunk - 3, buf0, ss0)
            gwait(pb, n_chunk - 2, buf1, gs1)
            scat(pb, n_chunk - 2, buf1, ss1)
            swait(pb, n_chunk - 3, buf0, ss0)
            gather(pb, n_chunk - 1, buf0, gs0)
            gwait(pb, n_chunk - 1, buf0, gs0)
            scat(pb, n_chunk - 1, buf0, ss0)
            swait(pb, n_chunk - 2, buf1, ss1)
            swait(pb, n_chunk - 1, buf0, ss0)

            if p + 1 < n_phase:
                pltpu.make_async_copy(src_hbm.at[w, p + 1], srcs.at[1 - pb],
                                      ssem).wait()
                pltpu.make_async_copy(dst_hbm.at[w, p + 1], dsts.at[1 - pb],
                                      ssem).wait()

        plsc.subcore_barrier()
        pltpu.sync_copy(acc.at[pl.ds(r0, rows_per_tile)],
                        out_hbm.at[c, pl.ds(r0, rows_per_tile)])

        @pl.when(s == 0)
        def _write_tail():
            pltpu.sync_copy(acc.at[pl.ds(tail_r0, tail_rows)],
                            out_hbm.at[c, pl.ds(tail_r0, tail_rows)])

    return agg


_agg_d128 = _make_agg(128)


def _mlp1_body(x_ref, a_ref, w1_ref, b1_ref, g_ref, bt_ref, w2_ref, b2_ref,
               o_ref):
    h = a_ref[0] + a_ref[1] - x_ref[...]
    t = jnp.dot(h, w1_ref[...], preferred_element_type=jnp.float32)
    t = t + b1_ref[...]
    mu = jnp.mean(t, axis=0, keepdims=True)
    var = jnp.mean(jnp.square(t - mu), axis=0, keepdims=True)
    t = (t - mu) * lax.rsqrt(var + 1e-5) * g_ref[...] + bt_ref[...]
    t = jnp.maximum(t, 0.0)
    t = jnp.dot(t, w2_ref[...], preferred_element_type=jnp.float32)
    o_ref[:, :64] = jnp.maximum(t + b2_ref[...], 0.0)
    o_ref[:, 64:] = jnp.zeros((_N, 64), jnp.float32)


def _mlp2_body(x_ref, a_ref, w1_ref, b1_ref, g_ref, bt_ref, w2_ref, b2_ref,
               fcw_ref, fcb_ref, emb_ref, out_ref):
    h = (a_ref[0] + a_ref[1] - x_ref[...])[:, :64]
    t = jnp.dot(h, w1_ref[...], preferred_element_type=jnp.float32)
    t = t + b1_ref[...]
    mu = jnp.mean(t, axis=0, keepdims=True)
    var = jnp.mean(jnp.square(t - mu), axis=0, keepdims=True)
    t = (t - mu) * lax.rsqrt(var + 1e-5) * g_ref[...] + bt_ref[...]
    t = jnp.maximum(t, 0.0)
    t = jnp.dot(t, w2_ref[...], preferred_element_type=jnp.float32)
    h2 = jnp.maximum(t + b2_ref[...], 0.0)
    emb_ref[...] = h2
    out_ref[...] = jnp.dot(h2, fcw_ref[...],
                           preferred_element_type=jnp.float32) + fcb_ref[...]


def kernel(x, edge_index, l1_w1, l1_b1, l1_bn_g, l1_bn_b, l1_w2, l1_b2,
           l2_w1, l2_b1, l2_bn_g, l2_bn_b, l2_w2, l2_b2, fc_w, fc_b):
    src = edge_index[0].reshape(_NC * _NS, 5, 25, _CH)
    dst = edge_index[1].reshape(_NC * _NS, 5, 25, _CH)

    agg1 = _agg_d128(x, src, dst)
    h1 = pl.pallas_call(
        _mlp1_body,
        out_shape=jax.ShapeDtypeStruct((_N, 128), jnp.float32),
    )(x, agg1, l1_w1, l1_b1, l1_bn_g, l1_bn_b, l1_w2, l1_b2)

    agg2 = _agg_d128(h1, src, dst)
    emb, out = pl.pallas_call(
        _mlp2_body,
        out_shape=(
            jax.ShapeDtypeStruct((_N, 32), jnp.float32),
            jax.ShapeDtypeStruct((_N, 64), jnp.float32),
        ),
    )(h1, agg2, l2_w1, l2_b1, l2_bn_g, l2_bn_b, l2_w2, l2_b2, fc_w, fc_b)

    return emb, out


# speedup vs baseline: 1.3166x; 1.3166x over previous
import functools

import jax
import jax.numpy as jnp
from jax import lax
from jax.experimental import pallas as pl
from jax.experimental.pallas import tpu as pltpu
from jax.experimental.pallas import tpu_sc as plsc

_N = 10000
_E = 320000
_NC = 2
_NS = 16
_CH = 80


def _make_agg(D, tc_tiling=True):
    rows_per_tile = 624
    tail_rows = _N - _NS * rows_per_tile
    tail_r0 = _NS * rows_per_tile
    n_phase = 5
    n_chunk = 25
    mesh = plsc.VectorSubcoreMesh(core_axis_name="c", subcore_axis_name="s")

    @functools.partial(
        pl.kernel,
        out_type=jax.ShapeDtypeStruct((_NC, _N, D), jnp.float32),
        mesh=mesh,
        scratch_types=[
            pltpu.VMEM((2, n_chunk, _CH), jnp.int32),
            pltpu.VMEM((2, n_chunk, _CH), jnp.int32),
            pltpu.VMEM((_CH, D), jnp.float32),
            pltpu.VMEM((_CH, D), jnp.float32),
            pltpu.VMEM_SHARED((_N, D), jnp.float32),
            pltpu.SemaphoreType.DMA,
            pltpu.SemaphoreType.DMA,
            pltpu.SemaphoreType.DMA,
        ],
        compiler_params=pltpu.CompilerParams(use_tc_tiling_on_sc=tc_tiling),
    )
    def agg(x_hbm, src_hbm, dst_hbm, out_hbm, srcs, dsts, buf0, buf1, acc,
            sem0, sem1, ssem):
        c = lax.axis_index("c")
        s = lax.axis_index("s")
        w = c * _NS + s
        r0 = pl.multiple_of(s * rows_per_tile, 8)
        pltpu.sync_copy(src_hbm.at[w, 0], srcs.at[0])
        pltpu.sync_copy(dst_hbm.at[w, 0], dsts.at[0])
        pltpu.sync_copy(x_hbm.at[pl.ds(r0, rows_per_tile)],
                        acc.at[pl.ds(r0, rows_per_tile)])

        @pl.when(s == 0)
        def _seed_tail():
            pltpu.sync_copy(x_hbm.at[pl.ds(tail_r0, tail_rows)],
                            acc.at[pl.ds(tail_r0, tail_rows)])

        plsc.subcore_barrier()

        for p in range(n_phase):
            pb = p % 2
            if p + 1 < n_phase:
                pltpu.async_copy(src_hbm.at[w, p + 1], srcs.at[1 - pb], ssem)
                pltpu.async_copy(dst_hbm.at[w, p + 1], dsts.at[1 - pb], ssem)

            pltpu.async_copy(x_hbm.at[srcs.at[pb, 0]], buf0, sem0)

            def body(i, carry, pb=pb):
                j = 2 * i
                pltpu.async_copy(x_hbm.at[srcs.at[pb, j + 1]], buf1, sem1)
                pltpu.make_async_copy(x_hbm.at[srcs.at[pb, j]], buf0,
                                      sem0).wait()
                pltpu.sync_copy(buf0, acc.at[dsts.at[pb, j]], add=True)
                pltpu.async_copy(x_hbm.at[srcs.at[pb, j + 2]], buf0, sem0)
                pltpu.make_async_copy(x_hbm.at[srcs.at[pb, j + 1]], buf1,
                                      sem1).wait()
                pltpu.sync_copy(buf1, acc.at[dsts.at[pb, j + 1]], add=True)
                return carry

            lax.fori_loop(0, (n_chunk - 1) // 2, body, 0)
            pltpu.make_async_copy(x_hbm.at[srcs.at[pb, n_chunk - 1]], buf0,
                                  sem0).wait()
            pltpu.sync_copy(buf0, acc.at[dsts.at[pb, n_chunk - 1]], add=True)

            if p + 1 < n_phase:
                pltpu.make_async_copy(src_hbm.at[w, p + 1], srcs.at[1 - pb],
                                      ssem).wait()
                pltpu.make_async_copy(dst_hbm.at[w, p + 1], dsts.at[1 - pb],
                                      ssem).wait()

        plsc.subcore_barrier()
        pltpu.sync_copy(acc.at[pl.ds(r0, rows_per_tile)],
                        out_hbm.at[c, pl.ds(r0, rows_per_tile)])

        @pl.when(s == 0)
        def _write_tail():
            pltpu.sync_copy(acc.at[pl.ds(tail_r0, tail_rows)],
                            out_hbm.at[c, pl.ds(tail_r0, tail_rows)])

    return agg


_agg_d128 = _make_agg(128)
_agg_d64 = _make_agg(64, tc_tiling=False)


def _mlp1_body(x_ref, a_ref, w1_ref, b1_ref, g_ref, bt_ref, w2_ref, b2_ref,
               o_ref):
    h = a_ref[0] + a_ref[1] - x_ref[...]
    t = jnp.dot(h, w1_ref[...], preferred_element_type=jnp.float32)
    t = t + b1_ref[...]
    mu = jnp.mean(t, axis=0, keepdims=True)
    var = jnp.mean(jnp.square(t - mu), axis=0, keepdims=True)
    t = (t - mu) * lax.rsqrt(var + 1e-5) * g_ref[...] + bt_ref[...]
    t = jnp.maximum(t, 0.0)
    t = jnp.dot(t, w2_ref[...], preferred_element_type=jnp.float32)
    o_ref[...] = jnp.maximum(t + b2_ref[...], 0.0)


def _mlp2_body(x_ref, a_ref, w1_ref, b1_ref, g_ref, bt_ref, w2_ref, b2_ref,
               fcw_ref, fcb_ref, emb_ref, out_ref):
    h = a_ref[0] + a_ref[1] - x_ref[...]
    t = jnp.dot(h, w1_ref[...], preferred_element_type=jnp.float32)
    t = t + b1_ref[...]
    mu = jnp.mean(t, axis=0, keepdims=True)
    var = jnp.mean(jnp.square(t - mu), axis=0, keepdims=True)
    t = (t - mu) * lax.rsqrt(var + 1e-5) * g_ref[...] + bt_ref[...]
    t = jnp.maximum(t, 0.0)
    t = jnp.dot(t, w2_ref[...], preferred_element_type=jnp.float32)
    h2 = jnp.maximum(t + b2_ref[...], 0.0)
    emb_ref[...] = h2
    out_ref[...] = jnp.dot(h2, fcw_ref[...],
                           preferred_element_type=jnp.float32) + fcb_ref[...]


def kernel(x, edge_index, l1_w1, l1_b1, l1_bn_g, l1_bn_b, l1_w2, l1_b2,
           l2_w1, l2_b1, l2_bn_g, l2_bn_b, l2_w2, l2_b2, fc_w, fc_b):
    src = edge_index[0].reshape(_NC * _NS, 5, 25, _CH)
    dst = edge_index[1].reshape(_NC * _NS, 5, 25, _CH)

    agg1 = _agg_d128(x, src, dst)
    h1 = pl.pallas_call(
        _mlp1_body,
        out_shape=jax.ShapeDtypeStruct((_N, 64), jnp.float32),
    )(x, agg1, l1_w1, l1_b1, l1_bn_g, l1_bn_b, l1_w2, l1_b2)

    agg2 = _agg_d64(h1, src, dst)
    emb, out = pl.pallas_call(
        _mlp2_body,
        out_shape=(
            jax.ShapeDtypeStruct((_N, 32), jnp.float32),
            jax.ShapeDtypeStruct((_N, 64), jnp.float32),
        ),
    )(h1, agg2, l2_w1, l2_b1, l2_bn_g, l2_bn_b, l2_w2, l2_b2, fc_w, fc_b)

    return emb, out


# speedup vs baseline: 1.4115x; 1.0721x over previous
import functools

import jax
import jax.numpy as jnp
from jax import lax
from jax.experimental import pallas as pl
from jax.experimental.pallas import tpu as pltpu
from jax.experimental.pallas import tpu_sc as plsc

_N = 10000
_E = 320000
_NC = 2
_NS = 16
_CH = 80
_NB = 5


def _make_agg(feature_split):
    rows_per_tile = 624
    tail_rows = _N - _NS * rows_per_tile
    tail_r0 = _NS * rows_per_tile
    n_workers = _NS if feature_split else _NC * _NS
    n_chunks = _E // (n_workers * _CH)
    mesh = plsc.VectorSubcoreMesh(core_axis_name="c", subcore_axis_name="s")

    @functools.partial(
        pl.kernel,
        out_type=jax.ShapeDtypeStruct((_NC, _N, 64), jnp.float32),
        mesh=mesh,
        scratch_types=[
            pltpu.VMEM((n_chunks, _CH), jnp.int32),
            pltpu.VMEM((n_chunks, _CH), jnp.int32),
        ] + [pltpu.VMEM((_CH, 64), jnp.float32) for _ in range(_NB)]
          + [pltpu.VMEM_SHARED((_N, 64), jnp.float32)]
          + [pltpu.SemaphoreType.DMA for _ in range(2 * _NB)],
        compiler_params=pltpu.CompilerParams(use_tc_tiling_on_sc=False),
    )
    def agg(x_hbm, src_hbm, dst_hbm, out_hbm, srcs, dsts, *rest):
        bufs = rest[:_NB]
        acc = rest[_NB]
        gsems = rest[_NB + 1:2 * _NB + 1]
        ssems = rest[2 * _NB + 1:]
        c = lax.axis_index("c")
        s = lax.axis_index("s")
        if feature_split:
            w = s
            table = x_hbm.at[c]
        else:
            w = c * _NS + s
            table = x_hbm
        r0 = pl.multiple_of(s * rows_per_tile, 8)
        pltpu.sync_copy(src_hbm.at[w], srcs)
        pltpu.sync_copy(dst_hbm.at[w], dsts)
        pltpu.sync_copy(table.at[pl.ds(r0, rows_per_tile)],
                        acc.at[pl.ds(r0, rows_per_tile)])

        @pl.when(s == 0)
        def _seed_tail():
            pltpu.sync_copy(table.at[pl.ds(tail_r0, tail_rows)],
                            acc.at[pl.ds(tail_r0, tail_rows)])

        plsc.subcore_barrier()

        def gather(j, k):
            return pltpu.async_copy(table.at[srcs.at[j]], bufs[k], gsems[k])

        def gwait(j, k):
            pltpu.make_async_copy(table.at[srcs.at[j]], bufs[k],
                                  gsems[k]).wait()

        def scat(j, k):
            return pltpu.async_copy(bufs[k], acc.at[dsts.at[j]], ssems[k],
                                    add=True)

        def swait(j, k):
            pltpu.make_async_copy(bufs[k], acc.at[dsts.at[j]],
                                  ssems[k]).wait()

        for k in range(_NB):
            gather(k, k)

        def body(i, carry):
            j = _NB * i
            for k in range(_NB):
                gwait(j + k, k)
                scat(j + k, k)
            for k in range(_NB):
                swait(j + k, k)

                @pl.when(j + k + _NB < n_chunks)
                def _refill(j=j, k=k):
                    gather(j + k + _NB, k)
            return carry

        lax.fori_loop(0, n_chunks // _NB, body, 0)
        plsc.subcore_barrier()
        pltpu.sync_copy(acc.at[pl.ds(r0, rows_per_tile)],
                        out_hbm.at[c, pl.ds(r0, rows_per_tile)])

        @pl.when(s == 0)
        def _write_tail():
            pltpu.sync_copy(acc.at[pl.ds(tail_r0, tail_rows)],
                            out_hbm.at[c, pl.ds(tail_r0, tail_rows)])

    return agg


_agg_l1 = _make_agg(feature_split=True)
_agg_l2 = _make_agg(feature_split=False)


def _mlp1_body(a_ref, w1_ref, b1_ref, g_ref, bt_ref, w2_ref, b2_ref, o_ref):
    h = jnp.concatenate([a_ref[0], a_ref[1]], axis=-1)
    t = jnp.dot(h, w1_ref[...], preferred_element_type=jnp.float32)
    t = t + b1_ref[...]
    mu = jnp.mean(t, axis=0, keepdims=True)
    var = jnp.mean(jnp.square(t - mu), axis=0, keepdims=True)
    t = (t - mu) * lax.rsqrt(var + 1e-5) * g_ref[...] + bt_ref[...]
    t = jnp.maximum(t, 0.0)
    t = jnp.dot(t, w2_ref[...], preferred_element_type=jnp.float32)
    o_ref[...] = jnp.maximum(t + b2_ref[...], 0.0)


def _mlp2_body(x_ref, a_ref, w1_ref, b1_ref, g_ref, bt_ref, w2_ref, b2_ref,
               fcw_ref, fcb_ref, emb_ref, out_ref):
    h = a_ref[0] + a_ref[1] - x_ref[...]
    t = jnp.dot(h, w1_ref[...], preferred_element_type=jnp.float32)
    t = t + b1_ref[...]
    mu = jnp.mean(t, axis=0, keepdims=True)
    var = jnp.mean(jnp.square(t - mu), axis=0, keepdims=True)
    t = (t - mu) * lax.rsqrt(var + 1e-5) * g_ref[...] + bt_ref[...]
    t = jnp.maximum(t, 0.0)
    t = jnp.dot(t, w2_ref[...], preferred_element_type=jnp.float32)
    h2 = jnp.maximum(t + b2_ref[...], 0.0)
    emb_ref[...] = h2
    out_ref[...] = jnp.dot(h2, fcw_ref[...],
                           preferred_element_type=jnp.float32) + fcb_ref[...]


def kernel(x, edge_index, l1_w1, l1_b1, l1_bn_g, l1_bn_b, l1_w2, l1_b2,
           l2_w1, l2_b1, l2_bn_g, l2_bn_b, l2_w2, l2_b2, fc_w, fc_b):
    src = edge_index[0]
    dst = edge_index[1]
    src_fs = src.reshape(_NS, _E // (_NS * _CH), _CH)
    dst_fs = dst.reshape(_NS, _E // (_NS * _CH), _CH)
    src_es = src.reshape(_NC * _NS, _E // (_NC * _NS * _CH), _CH)
    dst_es = dst.reshape(_NC * _NS, _E // (_NC * _NS * _CH), _CH)
    xs = jnp.stack([x[:, :64], x[:, 64:]])

    agg1 = _agg_l1(xs, src_fs, dst_fs)
    h1 = pl.pallas_call(
        _mlp1_body,
        out_shape=jax.ShapeDtypeStruct((_N, 64), jnp.float32),
    )(agg1, l1_w1, l1_b1, l1_bn_g, l1_bn_b, l1_w2, l1_b2)

    agg2 = _agg_l2(h1, src_es, dst_es)
    emb, out = pl.pallas_call(
        _mlp2_body,
        out_shape=(
            jax.ShapeDtypeStruct((_N, 32), jnp.float32),
            jax.ShapeDtypeStruct((_N, 64), jnp.float32),
        ),
    )(h1, agg2, l2_w1, l2_b1, l2_bn_g, l2_bn_b, l2_w2, l2_b2, fc_w, fc_b)

    return emb, out


# speedup vs baseline: 1.4197x; 1.0058x over previous
import functools

import jax
import jax.numpy as jnp
from jax import lax
from jax.experimental import pallas as pl
from jax.experimental.pallas import tpu as pltpu
from jax.experimental.pallas import tpu_sc as plsc

_N = 10000
_E = 320000
_NC = 2
_NS = 16
_CH = 80
_NB = 5


def _make_agg(feature_split):
    rows_per_tile = 624
    tail_rows = _N - _NS * rows_per_tile
    tail_r0 = _NS * rows_per_tile
    n_workers = _NS if feature_split else _NC * _NS
    n_chunks = _E // (n_workers * _CH)
    mesh = plsc.VectorSubcoreMesh(core_axis_name="c", subcore_axis_name="s")

    @functools.partial(
        pl.kernel,
        out_type=jax.ShapeDtypeStruct((_NC, _N, 64), jnp.float32),
        mesh=mesh,
        scratch_types=[
            pltpu.VMEM((n_chunks, _CH), jnp.int32),
            pltpu.VMEM((n_chunks, _CH), jnp.int32),
        ] + [pltpu.VMEM((_CH, 64), jnp.float32) for _ in range(_NB)]
          + [pltpu.VMEM_SHARED((_N, 64), jnp.float32)]
          + [pltpu.SemaphoreType.DMA for _ in range(2 * _NB)],
        compiler_params=pltpu.CompilerParams(use_tc_tiling_on_sc=False,
                                             disable_bounds_checks=True),
    )
    def agg(x_hbm, src_hbm, dst_hbm, out_hbm, srcs, dsts, *rest):
        bufs = rest[:_NB]
        acc = rest[_NB]
        gsems = rest[_NB + 1:2 * _NB + 1]
        ssems = rest[2 * _NB + 1:]
        c = lax.axis_index("c")
        s = lax.axis_index("s")
        if feature_split:
            table = x_hbm.at[c]
        else:
            table = x_hbm
        r0 = pl.multiple_of(s * rows_per_tile, 8)
        if feature_split:
            half = n_chunks // 2
            pltpu.sync_copy(src_hbm.at[2 * s], srcs.at[pl.ds(0, half)])
            pltpu.sync_copy(src_hbm.at[2 * s + 1], srcs.at[pl.ds(half, half)])
            pltpu.sync_copy(dst_hbm.at[2 * s], dsts.at[pl.ds(0, half)])
            pltpu.sync_copy(dst_hbm.at[2 * s + 1], dsts.at[pl.ds(half, half)])
        else:
            w = c * _NS + s
            pltpu.sync_copy(src_hbm.at[w], srcs)
            pltpu.sync_copy(dst_hbm.at[w], dsts)
        pltpu.sync_copy(table.at[pl.ds(r0, rows_per_tile)],
                        acc.at[pl.ds(r0, rows_per_tile)])

        @pl.when(s == 0)
        def _seed_tail():
            pltpu.sync_copy(table.at[pl.ds(tail_r0, tail_rows)],
                            acc.at[pl.ds(tail_r0, tail_rows)])

        plsc.subcore_barrier()

        def gather(j, k):
            return pltpu.async_copy(table.at[srcs.at[j]], bufs[k], gsems[k])

        def gwait(j, k):
            pltpu.make_async_copy(table.at[srcs.at[j]], bufs[k],
                                  gsems[k]).wait()

        def scat(j, k):
            return pltpu.async_copy(bufs[k], acc.at[dsts.at[j]], ssems[k],
                                    add=True)

        def swait(j, k):
            pltpu.make_async_copy(bufs[k], acc.at[dsts.at[j]],
                                  ssems[k]).wait()

        for k in range(_NB):
            gather(k, k)

        def body(i, carry):
            j = _NB * i
            for k in range(_NB):
                gwait(j + k, k)
                scat(j + k, k)
            for k in range(_NB):
                swait(j + k, k)

                @pl.when(j + k + _NB < n_chunks)
                def _refill(j=j, k=k):
                    gather(j + k + _NB, k)
            return carry

        lax.fori_loop(0, n_chunks // _NB, body, 0)
        plsc.subcore_barrier()
        pltpu.sync_copy(acc.at[pl.ds(r0, rows_per_tile)],
                        out_hbm.at[c, pl.ds(r0, rows_per_tile)])

        @pl.when(s == 0)
        def _write_tail():
            pltpu.sync_copy(acc.at[pl.ds(tail_r0, tail_rows)],
                            out_hbm.at[c, pl.ds(tail_r0, tail_rows)])

    return agg


_agg_l1 = _make_agg(feature_split=True)
_agg_l2 = _make_agg(feature_split=False)


def _mlp1_body(a_ref, w1_ref, b1_ref, g_ref, bt_ref, w2_ref, b2_ref, o_ref):
    h = jnp.concatenate([a_ref[0], a_ref[1]], axis=-1)
    t = jnp.dot(h, w1_ref[...], preferred_element_type=jnp.float32)
    t = t + b1_ref[...]
    mu = jnp.mean(t, axis=0, keepdims=True)
    var = jnp.mean(t * t, axis=0, keepdims=True) - mu * mu
    a = g_ref[...] * lax.rsqrt(var + 1e-5)
    b = bt_ref[...] - mu * a
    t = jnp.maximum(t * a + b, 0.0)
    t = jnp.dot(t, w2_ref[...], preferred_element_type=jnp.float32)
    o_ref[...] = jnp.maximum(t + b2_ref[...], 0.0)


def _mlp2_body(x_ref, a_ref, w1_ref, b1_ref, g_ref, bt_ref, w2_ref, b2_ref,
               fcw_ref, fcb_ref, emb_ref, out_ref):
    h = a_ref[0] + a_ref[1] - x_ref[...]
    t = jnp.dot(h, w1_ref[...], preferred_element_type=jnp.float32)
    t = t + b1_ref[...]
    mu = jnp.mean(t, axis=0, keepdims=True)
    var = jnp.mean(t * t, axis=0, keepdims=True) - mu * mu
    a = g_ref[...] * lax.rsqrt(var + 1e-5)
    b = bt_ref[...] - mu * a
    t = jnp.maximum(t * a + b, 0.0)
    t = jnp.dot(t, w2_ref[...], preferred_element_type=jnp.float32)
    h2 = jnp.maximum(t + b2_ref[...], 0.0)
    emb_ref[...] = h2
    out_ref[...] = jnp.dot(h2, fcw_ref[...],
                           preferred_element_type=jnp.float32) + fcb_ref[...]


def kernel(x, edge_index, l1_w1, l1_b1, l1_bn_g, l1_bn_b, l1_w2, l1_b2,
           l2_w1, l2_b1, l2_bn_g, l2_bn_b, l2_w2, l2_b2, fc_w, fc_b):
    src = edge_index[0].reshape(_NC * _NS, _E // (_NC * _NS * _CH), _CH)
    dst = edge_index[1].reshape(_NC * _NS, _E // (_NC * _NS * _CH), _CH)
    xs = jnp.stack([x[:, :64], x[:, 64:]])

    agg1 = _agg_l1(xs, src, dst)
    h1 = pl.pallas_call(
        _mlp1_body,
        out_shape=jax.ShapeDtypeStruct((_N, 64), jnp.float32),
    )(agg1, l1_w1, l1_b1, l1_bn_g, l1_bn_b, l1_w2, l1_b2)

    agg2 = _agg_l2(h1, src, dst)
    emb, out = pl.pallas_call(
        _mlp2_body,
        out_shape=(
            jax.ShapeDtypeStruct((_N, 32), jnp.float32),
            jax.ShapeDtypeStruct((_N, 64), jnp.float32),
        ),
    )(h1, agg2, l2_w1, l2_b1, l2_bn_g, l2_bn_b, l2_w2, l2_b2, fc_w, fc_b)

    return emb, out


# speedup vs baseline: 1.5663x; 1.1032x over previous
import functools

import jax
import jax.numpy as jnp
from jax import lax
from jax.experimental import pallas as pl
from jax.experimental.pallas import tpu as pltpu
from jax.experimental.pallas import tpu_sc as plsc

_N = 10000
_E = 320000
_NC = 2
_NS = 16
_CH = 80
_NB = 5


def _make_agg(feature_split):
    rows_per_tile = 624
    tail_rows = _N - _NS * rows_per_tile
    tail_r0 = _NS * rows_per_tile
    n_workers = _NS if feature_split else _NC * _NS
    n_chunks = _E // (n_workers * _CH)
    mesh = plsc.VectorSubcoreMesh(core_axis_name="c", subcore_axis_name="s")

    @functools.partial(
        pl.kernel,
        out_type=jax.ShapeDtypeStruct((_NC, _N, 64), jnp.float32),
        mesh=mesh,
        scratch_types=[
            pltpu.VMEM((n_chunks, _CH), jnp.int32),
            pltpu.VMEM((n_chunks, _CH), jnp.int32),
        ] + [pltpu.VMEM((_CH, 64), jnp.float32) for _ in range(_NB)]
          + [pltpu.VMEM_SHARED((_N, 64), jnp.float32)]
          + [pltpu.SemaphoreType.DMA for _ in range(2 * _NB)],
        compiler_params=pltpu.CompilerParams(use_tc_tiling_on_sc=False,
                                             disable_bounds_checks=True),
    )
    def agg(x_hbm, src_hbm, dst_hbm, out_hbm, srcs, dsts, *rest):
        bufs = rest[:_NB]
        acc = rest[_NB]
        gsems = rest[_NB + 1:2 * _NB + 1]
        ssems = rest[2 * _NB + 1:]
        c = lax.axis_index("c")
        s = lax.axis_index("s")
        if feature_split:
            table = x_hbm.at[c]
        else:
            table = x_hbm
        r0 = pl.multiple_of(s * rows_per_tile, 8)
        if feature_split:
            half = n_chunks // 2
            pltpu.sync_copy(src_hbm.at[2 * s], srcs.at[pl.ds(0, half)])
            pltpu.sync_copy(src_hbm.at[2 * s + 1], srcs.at[pl.ds(half, half)])
            pltpu.sync_copy(dst_hbm.at[2 * s], dsts.at[pl.ds(0, half)])
            pltpu.sync_copy(dst_hbm.at[2 * s + 1], dsts.at[pl.ds(half, half)])
        else:
            w = c * _NS + s
            pltpu.sync_copy(src_hbm.at[w], srcs)
            pltpu.sync_copy(dst_hbm.at[w], dsts)
        pltpu.sync_copy(table.at[pl.ds(r0, rows_per_tile)],
                        acc.at[pl.ds(r0, rows_per_tile)])

        @pl.when(s == 0)
        def _seed_tail():
            pltpu.sync_copy(table.at[pl.ds(tail_r0, tail_rows)],
                            acc.at[pl.ds(tail_r0, tail_rows)])

        plsc.subcore_barrier()

        def gather(j, k):
            return pltpu.async_copy(table.at[srcs.at[j]], bufs[k], gsems[k])

        def gwait(j, k):
            pltpu.make_async_copy(table.at[srcs.at[j]], bufs[k],
                                  gsems[k]).wait()

        def scat(j, k):
            return pltpu.async_copy(bufs[k], acc.at[dsts.at[j]], ssems[k],
                                    add=True)

        def swait(j, k):
            pltpu.make_async_copy(bufs[k], acc.at[dsts.at[j]],
                                  ssems[k]).wait()

        for k in range(_NB):
            gather(k, k)

        def body(i, carry):
            j = _NB * i
            for k in range(_NB):
                gwait(j + k, k)
                scat(j + k, k)
            for k in range(_NB):
                swait(j + k, k)

                @pl.when(j + k + _NB < n_chunks)
                def _refill(j=j, k=k):
                    gather(j + k + _NB, k)
            return carry

        lax.fori_loop(0, n_chunks // _NB, body, 0)
        plsc.subcore_barrier()
        pltpu.sync_copy(acc.at[pl.ds(r0, rows_per_tile)],
                        out_hbm.at[c, pl.ds(r0, rows_per_tile)])

        @pl.when(s == 0)
        def _write_tail():
            pltpu.sync_copy(acc.at[pl.ds(tail_r0, tail_rows)],
                            out_hbm.at[c, pl.ds(tail_r0, tail_rows)])

    return agg


_agg_l1 = _make_agg(feature_split=True)
_agg_l2 = _make_agg(feature_split=False)


def _packed_bn_relu(t, g, bt, width):
    m = jnp.mean(t, axis=0)
    m2 = jnp.mean(t * t, axis=0)
    mu = 0.5 * (m[:width] + m[width:])
    var = 0.5 * (m2[:width] + m2[width:]) - mu * mu
    a = g * lax.rsqrt(var + 1e-5)
    b = bt - mu * a
    aa = jnp.concatenate([a, a])
    bb = jnp.concatenate([b, b])
    return jnp.maximum(t * aa + bb, 0.0)


def _mlp1_body(a_ref, w1_ref, b1_ref, g_ref, bt_ref, w2_ref, b2_ref, o_ref):
    hq = jnp.concatenate([a_ref[0], a_ref[1]], axis=-1)
    t = jnp.dot(hq, w1_ref[...], preferred_element_type=jnp.float32)
    t = t + b1_ref[...]
    t = _packed_bn_relu(t, g_ref[...], bt_ref[...], 128)
    t = jnp.dot(t, w2_ref[...], preferred_element_type=jnp.float32)
    o_ref[...] = jnp.maximum(t + b2_ref[...], 0.0)


def _mlp2_body(x_ref, a_ref, w1_ref, b1_ref, g_ref, bt_ref, w2_ref, b2_ref,
               fcw_ref, fcb_ref, emb_ref, out_ref):
    hp = a_ref[0] + a_ref[1] - x_ref[...]
    t = jnp.dot(hp, w1_ref[...], preferred_element_type=jnp.float32)
    t = t + b1_ref[...]
    t = _packed_bn_relu(t, g_ref[...], bt_ref[...], 64)
    t = jnp.dot(t, w2_ref[...], preferred_element_type=jnp.float32)
    h2 = jnp.maximum(t + b2_ref[...], 0.0)
    emb_ref[...] = h2
    out_ref[...] = jnp.dot(h2, fcw_ref[...],
                           preferred_element_type=jnp.float32) + fcb_ref[...]


def _bdiag(w):
    z = jnp.zeros_like(w)
    return jnp.concatenate([jnp.concatenate([w, z], 1),
                            jnp.concatenate([z, w], 1)], 0)


def kernel(x, edge_index, l1_w1, l1_b1, l1_bn_g, l1_bn_b, l1_w2, l1_b2,
           l2_w1, l2_b1, l2_bn_g, l2_bn_b, l2_w2, l2_b2, fc_w, fc_b):
    src = edge_index[0].reshape(_NC * _NS, _E // (_NC * _NS * _CH), _CH)
    dst = edge_index[1].reshape(_NC * _NS, _E // (_NC * _NS * _CH), _CH)
    xs = jnp.stack([x[:, :64], x[:, 64:]])

    w1p = jnp.concatenate([_bdiag(l1_w1[:64]), _bdiag(l1_w1[64:])], 0)
    b1p = jnp.concatenate([l1_b1, l1_b1])
    w2p = _bdiag(l1_w2)
    b2p = jnp.concatenate([l1_b2, l1_b2])
    v1p = _bdiag(l2_w1)
    c1p = jnp.concatenate([l2_b1, l2_b1])
    v2p = _bdiag(l2_w2)
    c2p = jnp.concatenate([l2_b2, l2_b2])
    vfp = _bdiag(fc_w)
    fbp = jnp.concatenate([fc_b, fc_b])

    agg1 = _agg_l1(xs, src, dst)
    h1p = pl.pallas_call(
        _mlp1_body,
        out_shape=jax.ShapeDtypeStruct((_N // 2, 128), jnp.float32),
    )(agg1.reshape(_NC, _N // 2, 128), w1p, b1p, l1_bn_g, l1_bn_b, w2p, b2p)

    agg2 = _agg_l2(h1p.reshape(_N, 64), src, dst)
    emb_p, out_p = pl.pallas_call(
        _mlp2_body,
        out_shape=(
            jax.ShapeDtypeStruct((_N // 2, 64), jnp.float32),
            jax.ShapeDtypeStruct((_N // 2, 128), jnp.float32),
        ),
    )(h1p, agg2.reshape(_NC, _N // 2, 128), v1p, c1p, l2_bn_g, l2_bn_b,
      v2p, c2p, vfp, fbp)

    return emb_p.reshape(_N, 32), out_p.reshape(_N, 64)


# speedup vs baseline: 1.6117x; 1.0290x over previous
import functools

import jax
import jax.numpy as jnp
from jax import lax
from jax.experimental import pallas as pl
from jax.experimental.pallas import tpu as pltpu
from jax.experimental.pallas import tpu_sc as plsc

_N = 10000
_E = 320000
_NC = 2
_NS = 16
_CH = 80
_NB = 5


def _make_agg(feature_split):
    rows_per_tile = 624
    tail_rows = _N - _NS * rows_per_tile
    tail_r0 = _NS * rows_per_tile
    n_workers = _NS if feature_split else _NC * _NS
    n_chunks = _E // (n_workers * _CH)
    mesh = plsc.VectorSubcoreMesh(core_axis_name="c", subcore_axis_name="s")

    @functools.partial(
        pl.kernel,
        out_type=jax.ShapeDtypeStruct((_NC, _N, 64), jnp.float32),
        mesh=mesh,
        scratch_types=[
            pltpu.VMEM((n_chunks, _CH), jnp.int32),
            pltpu.VMEM((n_chunks, _CH), jnp.int32),
            pltpu.VMEM((_CH,), jnp.int32),
        ] + [pltpu.VMEM((_CH, 64), jnp.float32) for _ in range(_NB)]
          + [pltpu.VMEM_SHARED((_N, 64), jnp.float32)]
          + [pltpu.SemaphoreType.DMA for _ in range(2 * _NB)],
        compiler_params=pltpu.CompilerParams(use_tc_tiling_on_sc=False,
                                             disable_bounds_checks=True),
    )
    def agg(x_hbm, src_hbm, dst_hbm, out_hbm, srcs, dsts, sidx, *rest):
        bufs = rest[:_NB]
        acc = rest[_NB]
        gsems = rest[_NB + 1:2 * _NB + 1]
        ssems = rest[2 * _NB + 1:]
        c = lax.axis_index("c")
        s = lax.axis_index("s")
        table = x_hbm
        r0 = pl.multiple_of(s * rows_per_tile, 8)
        if feature_split:
            half = n_chunks // 2
            pltpu.sync_copy(src_hbm.at[2 * s], srcs.at[pl.ds(0, half)])
            pltpu.sync_copy(src_hbm.at[2 * s + 1], srcs.at[pl.ds(half, half)])
            pltpu.sync_copy(dst_hbm.at[2 * s], dsts.at[pl.ds(0, half)])
            pltpu.sync_copy(dst_hbm.at[2 * s + 1], dsts.at[pl.ds(half, half)])
        else:
            w = c * _NS + s
            pltpu.sync_copy(src_hbm.at[w], srcs)
            pltpu.sync_copy(dst_hbm.at[w], dsts)

        if feature_split:
            def xform(j):
                for k in range(_CH // 16):
                    v = srcs[j, pl.ds(16 * k, 16)]
                    srcs[j, pl.ds(16 * k, 16)] = 2 * v + c

            def seed(base, cnt):
                for k in range(_CH // 16):
                    sidx[pl.ds(16 * k, 16)] = (
                        2 * (base + 16 * k + lax.iota(jnp.int32, 16)) + c)
                pltpu.async_copy(table.at[sidx.at[pl.ds(0, cnt)]],
                                 bufs[0].at[pl.ds(0, cnt)], gsems[0]).wait()
                pltpu.sync_copy(bufs[0].at[pl.ds(0, cnt)],
                                acc.at[pl.ds(base, cnt)])

            def seed_loop(i, carry):
                seed(r0 + i * _CH, _CH)
                return carry

            lax.fori_loop(0, rows_per_tile // _CH, seed_loop, 0)
            seed(r0 + rows_per_tile - (rows_per_tile % _CH or _CH),
                 rows_per_tile % _CH or _CH)

            @pl.when(s == 0)
            def _seed_tail():
                seed(tail_r0, tail_rows)
        else:
            pltpu.sync_copy(table.at[pl.ds(r0, rows_per_tile)],
                            acc.at[pl.ds(r0, rows_per_tile)])

            @pl.when(s == 0)
            def _seed_tail():
                pltpu.sync_copy(table.at[pl.ds(tail_r0, tail_rows)],
                                acc.at[pl.ds(tail_r0, tail_rows)])

        plsc.subcore_barrier()

        def gather(j, k):
            return pltpu.async_copy(table.at[srcs.at[j]], bufs[k], gsems[k])

        def gwait(j, k):
            pltpu.make_async_copy(table.at[srcs.at[j]], bufs[k],
                                  gsems[k]).wait()

        def scat(j, k):
            return pltpu.async_copy(bufs[k], acc.at[dsts.at[j]], ssems[k],
                                    add=True)

        def swait(j, k):
            pltpu.make_async_copy(bufs[k], acc.at[dsts.at[j]],
                                  ssems[k]).wait()

        for k in range(_NB):
            if feature_split:
                xform(k)
            gather(k, k)

        def body(i, carry):
            j = _NB * i
            for k in range(_NB):
                gwait(j + k, k)
                scat(j + k, k)
            for k in range(_NB):
                swait(j + k, k)

                @pl.when(j + k + _NB < n_chunks)
                def _refill(j=j, k=k):
                    if feature_split:
                        xform(j + k + _NB)
                    gather(j + k + _NB, k)
            return carry

        lax.fori_loop(0, n_chunks // _NB, body, 0)
        plsc.subcore_barrier()
        pltpu.sync_copy(acc.at[pl.ds(r0, rows_per_tile)],
                        out_hbm.at[c, pl.ds(r0, rows_per_tile)])

        @pl.when(s == 0)
        def _write_tail():
            pltpu.sync_copy(acc.at[pl.ds(tail_r0, tail_rows)],
                            out_hbm.at[c, pl.ds(tail_r0, tail_rows)])

    return agg


_agg_l1 = _make_agg(feature_split=True)
_agg_l2 = _make_agg(feature_split=False)


def _packed_bn_relu(t, g, bt, width):
    m = jnp.mean(t, axis=0)
    m2 = jnp.mean(t * t, axis=0)
    mu = 0.5 * (m[:width] + m[width:])
    var = 0.5 * (m2[:width] + m2[width:]) - mu * mu
    a = g * lax.rsqrt(var + 1e-5)
    b = bt - mu * a
    aa = jnp.concatenate([a, a])
    bb = jnp.concatenate([b, b])
    return jnp.maximum(t * aa + bb, 0.0)


def _pdot(tp, w):
    k = w.shape[0]
    le = jnp.dot(tp[:, :k], w, preferred_element_type=jnp.float32)
    ro = jnp.dot(tp[:, k:], w, preferred_element_type=jnp.float32)
    return jnp.concatenate([le, ro], axis=-1)


def _mlp1_body(a_ref, w1_ref, b1_ref, g_ref, bt_ref, w2_ref, b2_ref, o_ref):
    a0 = a_ref[0]
    a1 = a_ref[1]
    w1a = w1_ref[:64, :]
    w1b = w1_ref[64:, :]
    t_even = (jnp.dot(a0[:, :64], w1a, preferred_element_type=jnp.float32) +
              jnp.dot(a1[:, :64], w1b, preferred_element_type=jnp.float32))
    t_odd = (jnp.dot(a0[:, 64:], w1a, preferred_element_type=jnp.float32) +
             jnp.dot(a1[:, 64:], w1b, preferred_element_type=jnp.float32))
    b1 = b1_ref[...]
    t = jnp.concatenate([t_even + b1, t_odd + b1], axis=-1)
    t = _packed_bn_relu(t, g_ref[...], bt_ref[...], 128)
    t = _pdot(t, w2_ref[...])
    b2 = b2_ref[...]
    o_ref[...] = jnp.maximum(t + jnp.concatenate([b2, b2]), 0.0)


def _mlp2_body(x_ref, a_ref, w1_ref, b1_ref, g_ref, bt_ref, w2_ref, b2_ref,
               fcw_ref, fcb_ref, emb_ref, out_ref):
    hp = a_ref[0] + a_ref[1] - x_ref[...]
    b1 = b1_ref[...]
    t = _pdot(hp, w1_ref[...]) + jnp.concatenate([b1, b1])
    t = _packed_bn_relu(t, g_ref[...], bt_ref[...], 64)
    b2 = b2_ref[...]
    t = _pdot(t, w2_ref[...]) + jnp.concatenate([b2, b2])
    h2 = jnp.maximum(t, 0.0)
    emb_ref[...] = h2
    fcb = fcb_ref[...]
    out_ref[...] = _pdot(h2, fcw_ref[...]) + jnp.concatenate([fcb, fcb])


def kernel(x, edge_index, l1_w1, l1_b1, l1_bn_g, l1_bn_b, l1_w2, l1_b2,
           l2_w1, l2_b1, l2_bn_g, l2_bn_b, l2_w2, l2_b2, fc_w, fc_b):
    src = edge_index[0].reshape(_NC * _NS, _E // (_NC * _NS * _CH), _CH)
    dst = edge_index[1].reshape(_NC * _NS, _E // (_NC * _NS * _CH), _CH)

    agg1 = _agg_l1(x.reshape(2 * _N, 64), src, dst)
    h1p = pl.pallas_call(
        _mlp1_body,
        out_shape=jax.ShapeDtypeStruct((_N // 2, 128), jnp.float32),
    )(agg1.reshape(_NC, _N // 2, 128), l1_w1, l1_b1, l1_bn_g, l1_bn_b,
      l1_w2, l1_b2)

    agg2 = _agg_l2(h1p.reshape(_N, 64), src, dst)
    emb_p, out_p = pl.pallas_call(
        _mlp2_body,
        out_shape=(
            jax.ShapeDtypeStruct((_N // 2, 64), jnp.float32),
            jax.ShapeDtypeStruct((_N // 2, 128), jnp.float32),
        ),
    )(h1p, agg2.reshape(_NC, _N // 2, 128), l2_w1, l2_b1, l2_bn_g, l2_bn_b,
      l2_w2, l2_b2, fc_w, fc_b)

    return emb_p.reshape(_N, 32), out_p.reshape(_N, 64)


# speedup vs baseline: 1.6483x; 1.0227x over previous
import functools

import jax
import jax.numpy as jnp
from jax import lax
from jax.experimental import pallas as pl
from jax.experimental.pallas import tpu as pltpu
from jax.experimental.pallas import tpu_sc as plsc

_N = 10000
_E = 320000
_NC = 2
_NS = 16
_CH = 80
_NB = 5


def _make_agg(feature_split):
    rows_per_tile = 624
    tail_rows = _N - _NS * rows_per_tile
    tail_r0 = _NS * rows_per_tile
    n_workers = _NS if feature_split else _NC * _NS
    n_chunks = _E // (n_workers * _CH)
    mesh = plsc.VectorSubcoreMesh(core_axis_name="c", subcore_axis_name="s")

    @functools.partial(
        pl.kernel,
        out_type=jax.ShapeDtypeStruct((_NC, _N, 64), jnp.float32),
        mesh=mesh,
        scratch_types=[
            pltpu.VMEM((n_chunks, _CH), jnp.int32),
            pltpu.VMEM((n_chunks, _CH), jnp.int32),
            pltpu.VMEM((8, _CH), jnp.int32),
        ] + [pltpu.VMEM((_CH, 64), jnp.float32) for _ in range(_NB)]
          + [pltpu.VMEM_SHARED((_N, 64), jnp.float32)]
          + [pltpu.SemaphoreType.DMA for _ in range(2 * _NB)],
        compiler_params=pltpu.CompilerParams(use_tc_tiling_on_sc=False,
                                             disable_bounds_checks=True),
    )
    def agg(x_hbm, src_hbm, dst_hbm, out_hbm, srcs, dsts, sidx, *rest):
        bufs = rest[:_NB]
        acc = rest[_NB]
        gsems = rest[_NB + 1:2 * _NB + 1]
        ssems = rest[2 * _NB + 1:]
        c = lax.axis_index("c")
        s = lax.axis_index("s")
        table = x_hbm
        r0 = pl.multiple_of(s * rows_per_tile, 8)
        if feature_split:
            half = n_chunks // 2
            pltpu.sync_copy(src_hbm.at[2 * s], srcs.at[pl.ds(0, half)])
            pltpu.sync_copy(src_hbm.at[2 * s + 1], srcs.at[pl.ds(half, half)])
            pltpu.sync_copy(dst_hbm.at[2 * s], dsts.at[pl.ds(0, half)])
            pltpu.sync_copy(dst_hbm.at[2 * s + 1], dsts.at[pl.ds(half, half)])
        else:
            w = c * _NS + s
            pltpu.sync_copy(src_hbm.at[w], srcs)
            pltpu.sync_copy(dst_hbm.at[w], dsts)

        if feature_split:
            def xform(j):
                for k in range(_CH // 16):
                    v = srcs[j, pl.ds(16 * k, 16)]
                    srcs[j, pl.ds(16 * k, 16)] = 2 * v + c

            sizes = [_CH] * 7 + [rows_per_tile - 7 * _CH]

            def sfill(t):
                for k in range(_CH // 16):
                    sidx[t, pl.ds(16 * k, 16)] = (
                        2 * (r0 + _CH * t + 16 * k +
                             lax.iota(jnp.int32, 16)) + c)

            def sgather(t, k):
                return pltpu.async_copy(
                    table.at[sidx.at[t, pl.ds(0, sizes[t])]],
                    bufs[k].at[pl.ds(0, sizes[t])], gsems[k])

            def sgwait(t, k):
                pltpu.make_async_copy(
                    table.at[sidx.at[t, pl.ds(0, sizes[t])]],
                    bufs[k].at[pl.ds(0, sizes[t])], gsems[k]).wait()

            for t in range(_NB):
                sfill(t)
                sgather(t, t)
            for t in range(len(sizes)):
                k = t % _NB
                sgwait(t, k)
                pltpu.sync_copy(bufs[k].at[pl.ds(0, sizes[t])],
                                acc.at[pl.ds(r0 + _CH * t, sizes[t])])
                if t + _NB < len(sizes):
                    sfill(t + _NB)
                    sgather(t + _NB, k)

            @pl.when(s == 0)
            def _seed_tail():
                for k in range(_CH // 16):
                    sidx[0, pl.ds(16 * k, 16)] = (
                        2 * (tail_r0 + 16 * k + lax.iota(jnp.int32, 16)) + c)
                pltpu.async_copy(table.at[sidx.at[0, pl.ds(0, tail_rows)]],
                                 bufs[0].at[pl.ds(0, tail_rows)],
                                 gsems[0]).wait()
                pltpu.sync_copy(bufs[0].at[pl.ds(0, tail_rows)],
                                acc.at[pl.ds(tail_r0, tail_rows)])
        else:
            pltpu.sync_copy(table.at[pl.ds(r0, rows_per_tile)],
                            acc.at[pl.ds(r0, rows_per_tile)])

            @pl.when(s == 0)
            def _seed_tail():
                pltpu.sync_copy(table.at[pl.ds(tail_r0, tail_rows)],
                                acc.at[pl.ds(tail_r0, tail_rows)])

        plsc.subcore_barrier()

        def gather(j, k):
            return pltpu.async_copy(table.at[srcs.at[j]], bufs[k], gsems[k])

        def gwait(j, k):
            pltpu.make_async_copy(table.at[srcs.at[j]], bufs[k],
                                  gsems[k]).wait()

        def scat(j, k):
            return pltpu.async_copy(bufs[k], acc.at[dsts.at[j]], ssems[k],
                                    add=True)

        def swait(j, k):
            pltpu.make_async_copy(bufs[k], acc.at[dsts.at[j]],
                                  ssems[k]).wait()

        for k in range(_NB):
            if feature_split:
                xform(k)
            gather(k, k)

        def body(i, carry):
            j = _NB * i
            for k in range(_NB):
                gwait(j + k, k)
                scat(j + k, k)
            for k in range(_NB):
                swait(j + k, k)

                @pl.when(j + k + _NB < n_chunks)
                def _refill(j=j, k=k):
                    if feature_split:
                        xform(j + k + _NB)
                    gather(j + k + _NB, k)
            return carry

        lax.fori_loop(0, n_chunks // _NB, body, 0)
        plsc.subcore_barrier()
        pltpu.sync_copy(acc.at[pl.ds(r0, rows_per_tile)],
                        out_hbm.at[c, pl.ds(r0, rows_per_tile)])

        @pl.when(s == 0)
        def _write_tail():
            pltpu.sync_copy(acc.at[pl.ds(tail_r0, tail_rows)],
                            out_hbm.at[c, pl.ds(tail_r0, tail_rows)])

    return agg


_agg_l1 = _make_agg(feature_split=True)
_agg_l2 = _make_agg(feature_split=False)


def _packed_bn_relu(t, g, bt, width):
    m = jnp.mean(t, axis=0)
    m2 = jnp.mean(t * t, axis=0)
    mu = 0.5 * (m[:width] + m[width:])
    var = 0.5 * (m2[:width] + m2[width:]) - mu * mu
    a = g * lax.rsqrt(var + 1e-5)
    b = bt - mu * a
    aa = jnp.concatenate([a, a])
    bb = jnp.concatenate([b, b])
    return jnp.maximum(t * aa + bb, 0.0)


def _pdot(tp, w):
    k = w.shape[0]
    le = jnp.dot(tp[:, :k], w, preferred_element_type=jnp.float32)
    ro = jnp.dot(tp[:, k:], w, preferred_element_type=jnp.float32)
    return jnp.concatenate([le, ro], axis=-1)


def _mlp1_body(a_ref, w1_ref, b1_ref, g_ref, bt_ref, w2_ref, b2_ref, o_ref):
    a0 = a_ref[0]
    a1 = a_ref[1]
    w1a = w1_ref[:64, :]
    w1b = w1_ref[64:, :]
    t_even = (jnp.dot(a0[:, :64], w1a, preferred_element_type=jnp.float32) +
              jnp.dot(a1[:, :64], w1b, preferred_element_type=jnp.float32))
    t_odd = (jnp.dot(a0[:, 64:], w1a, preferred_element_type=jnp.float32) +
             jnp.dot(a1[:, 64:], w1b, preferred_element_type=jnp.float32))
    b1 = b1_ref[...]
    t = jnp.concatenate([t_even + b1, t_odd + b1], axis=-1)
    t = _packed_bn_relu(t, g_ref[...], bt_ref[...], 128)
    t = _pdot(t, w2_ref[...])
    b2 = b2_ref[...]
    o_ref[...] = jnp.maximum(t + jnp.concatenate([b2, b2]), 0.0)


def _mlp2_body(x_ref, a_ref, w1_ref, b1_ref, g_ref, bt_ref, w2_ref, b2_ref,
               fcw_ref, fcb_ref, emb_ref, out_ref):
    hp = a_ref[0] + a_ref[1] - x_ref[...]
    b1 = b1_ref[...]
    t = _pdot(hp, w1_ref[...]) + jnp.concatenate([b1, b1])
    t = _packed_bn_relu(t, g_ref[...], bt_ref[...], 64)
    b2 = b2_ref[...]
    t = _pdot(t, w2_ref[...]) + jnp.concatenate([b2, b2])
    h2 = jnp.maximum(t, 0.0)
    emb_ref[...] = h2
    fcb = fcb_ref[...]
    out_ref[...] = _pdot(h2, fcw_ref[...]) + jnp.concatenate([fcb, fcb])


def kernel(x, edge_index, l1_w1, l1_b1, l1_bn_g, l1_bn_b, l1_w2, l1_b2,
           l2_w1, l2_b1, l2_bn_g, l2_bn_b, l2_w2, l2_b2, fc_w, fc_b):
    src = edge_index[0].reshape(_NC * _NS, _E // (_NC * _NS * _CH), _CH)
    dst = edge_index[1].reshape(_NC * _NS, _E // (_NC * _NS * _CH), _CH)

    agg1 = _agg_l1(x.reshape(2 * _N, 64), src, dst)
    h1p = pl.pallas_call(
        _mlp1_body,
        out_shape=jax.ShapeDtypeStruct((_N // 2, 128), jnp.float32),
    )(agg1.reshape(_NC, _N // 2, 128), l1_w1, l1_b1, l1_bn_g, l1_bn_b,
      l1_w2, l1_b2)

    agg2 = _agg_l2(h1p.reshape(_N, 64), src, dst)
    emb_p, out_p = pl.pallas_call(
        _mlp2_body,
        out_shape=(
            jax.ShapeDtypeStruct((_N // 2, 64), jnp.float32),
            jax.ShapeDtypeStruct((_N // 2, 128), jnp.float32),
        ),
    )(h1p, agg2.reshape(_NC, _N // 2, 128), l2_w1, l2_b1, l2_bn_g, l2_bn_b,
      l2_w2, l2_b2, fc_w, fc_b)

    return emb_p.reshape(_N, 32), out_p.reshape(_N, 64)
